# Initial kernel scaffold; baseline (speedup 1.0000x reference)
import jax
import jax.numpy as jnp
from jax.experimental import pallas as pl


def kernel(static_node_feat, src, dst, neg, time, nbr_nids, nbr_times, nbr_feats, src_nbr_idx, dst_nbr_idx, neg_nbr_idx, t2v_w, t2v_b, P, W1, b1, W2, b2, Wself, Wd1, bd1, Wd2, bd2):
    raise NotImplementedError("write your pallas kernel here")


# speedup vs baseline: 2.3866x; 2.3866x over previous
import functools

import jax
import jax.numpy as jnp
from jax import lax
from jax.experimental import pallas as pl
from jax.experimental.pallas import tpu as pltpu
from jax.experimental.pallas import tpu_sc as plsc

N = 100000
B = 1024
K = 32
F = 128
EF = 16
T = 100
RP = 16
H = 128

S3 = 3 * B
RT = S3 * K
NW = 32
S_PT = S3 // NW

_SC_MESH = dict(core_axis_name="c", subcore_axis_name="s")


def _wid():
    return lax.axis_index("s") * 2 + lax.axis_index("c")


def _sc_wide_body(static_hbm, nn_hbm, seeds_hbm, g_out, ss_out,
                  nnv, sv, ssv, gb, sem):
    wid = _wid()
    base_s = wid * S_PT

    pltpu.sync_copy(nn_hbm.at[pl.ds(wid * (K * S_PT), K * S_PT)], nnv)
    pltpu.sync_copy(seeds_hbm.at[pl.ds(base_s, S_PT)], sv)

    pltpu.async_copy(static_hbm.at[sv], ssv, sem).wait()
    pltpu.sync_copy(ssv, ss_out.at[pl.ds(base_s, S_PT)])

    def per_j(j, carry):
        idxs = nnv.at[pl.ds(j * S_PT, S_PT)]
        pltpu.async_copy(static_hbm.at[idxs], gb, sem).wait()
        pltpu.sync_copy(gb, g_out.at[pl.ds(j * S3 + base_s, S_PT)])
        return carry

    lax.fori_loop(0, K, per_j, 0)


@functools.lru_cache(maxsize=1)
def _sc_wide_fn():
    return functools.partial(
        pl.kernel,
        out_type=[
            jax.ShapeDtypeStruct((RT, F), jnp.float32),
            jax.ShapeDtypeStruct((S3, F), jnp.float32),
        ],
        mesh=plsc.VectorSubcoreMesh(**_SC_MESH),
        scratch_types=[
            pltpu.VMEM((K * S_PT,), jnp.int32),
            pltpu.VMEM((S_PT,), jnp.int32),
            pltpu.VMEM((S_PT, F), jnp.float32),
            pltpu.VMEM((S_PT, F), jnp.float32),
            pltpu.SemaphoreType.DMA,
        ],
    )(_sc_wide_body)


def _sc_narrow_body(p_hbm, nfsrc_hbm, nn_hbm, ie_hbm, seeds_hbm, idx_hbm,
                    nt_hbm, pn_out, nf_out, ps_out, nt_out,
                    nnv, iev, sv, iv, psv, ntv, pnb, nfb, sem):
    wid = _wid()
    base_s = wid * S_PT

    pltpu.sync_copy(nn_hbm.at[pl.ds(wid * (K * S_PT), K * S_PT)], nnv)
    pltpu.sync_copy(ie_hbm.at[pl.ds(wid * (K * S_PT), K * S_PT)], iev)
    pltpu.sync_copy(seeds_hbm.at[pl.ds(base_s, S_PT)], sv)
    pltpu.sync_copy(idx_hbm.at[pl.ds(base_s, S_PT)], iv)

    pltpu.async_copy(p_hbm.at[sv], psv, sem).wait()
    pltpu.sync_copy(psv, ps_out.at[pl.ds(base_s, S_PT)])
    pltpu.async_copy(nt_hbm.at[iv], ntv, sem).wait()
    pltpu.sync_copy(ntv, nt_out.at[pl.ds(base_s, S_PT)])

    def per_j(j, carry):
        idxs = nnv.at[pl.ds(j * S_PT, S_PT)]
        pltpu.async_copy(p_hbm.at[idxs], pnb, sem).wait()
        pltpu.sync_copy(pnb, pn_out.at[pl.ds(j * S3 + base_s, S_PT)])
        idxe = iev.at[pl.ds(j * S_PT, S_PT)]
        pltpu.async_copy(nfsrc_hbm.at[idxe], nfb, sem).wait()
        pltpu.sync_copy(nfb, nf_out.at[pl.ds(j * S3 + base_s, S_PT)])
        return carry

    lax.fori_loop(0, K, per_j, 0)


@functools.lru_cache(maxsize=1)
def _sc_narrow_fn():
    return functools.partial(
        pl.kernel,
        out_type=[
            jax.ShapeDtypeStruct((K * S3, RP), jnp.float32),
            jax.ShapeDtypeStruct((K * S3, EF), jnp.float32),
            jax.ShapeDtypeStruct((S3, RP), jnp.float32),
            jax.ShapeDtypeStruct((S3, K), jnp.float32),
        ],
        mesh=plsc.VectorSubcoreMesh(**_SC_MESH),
        compiler_params=pltpu.CompilerParams(use_tc_tiling_on_sc=False),
        scratch_types=[
            pltpu.VMEM((K * S_PT,), jnp.int32),
            pltpu.VMEM((K * S_PT,), jnp.int32),
            pltpu.VMEM((S_PT,), jnp.int32),
            pltpu.VMEM((S_PT,), jnp.int32),
            pltpu.VMEM((S_PT, RP), jnp.float32),
            pltpu.VMEM((S_PT, K), jnp.float32),
            pltpu.VMEM((S_PT, RP), jnp.float32),
            pltpu.VMEM((S_PT, EF), jnp.float32),
            pltpu.SemaphoreType.DMA,
        ],
    )(_sc_narrow_body)


SB = 256
RB = SB * K
NBLK = S3 // SB


def _encode_body(g_ref, nf_ref, pn_ref, nt_ref, ps_ref, t2_ref, ss_ref,
                 w1f_ref, w1e_ref, w1t_ref, w1r_ref, b1_ref, w2_ref, b2_ref,
                 wself_ref, tw_ref, tb_ref, z_ref):
    nt = nt_ref[...]
    t2 = t2_ref[...]
    ps = ps_ref[...]
    g3 = g_ref[...]
    nf3 = nf_ref[...]
    pn3 = pn_ref[...]

    dtm = t2 - nt
    dt_full = jnp.concatenate([dtm[:, j:j + 1] for j in range(K)], axis=0)
    te = jnp.cos(dt_full * tw_ref[...] + tb_ref[...])
    g2 = jnp.concatenate([g3[j] for j in range(K)], axis=0)
    nf2 = jnp.concatenate([nf3[j] for j in range(K)], axis=0)
    rp = jnp.concatenate(
        [jnp.sum(pn3[j] * ps, axis=-1, keepdims=True) for j in range(K)],
        axis=0)

    pre = (jnp.dot(g2, w1f_ref[...], preferred_element_type=jnp.float32)
           + jnp.dot(nf2, w1e_ref[...], preferred_element_type=jnp.float32)
           + jnp.dot(te, w1t_ref[...], preferred_element_type=jnp.float32)
           + rp * w1r_ref[...]
           + b1_ref[...])
    h = jnp.maximum(pre, 0.0)
    acc = h[:SB]
    for j in range(1, K):
        acc = acc + h[j * SB:(j + 1) * SB]
    m = acc * (1.0 / K)
    z = (jnp.dot(m, w2_ref[...], preferred_element_type=jnp.float32)
         + b2_ref[...]
         + jnp.dot(ss_ref[...], wself_ref[...], preferred_element_type=jnp.float32))
    z_ref[...] = z


def _decode_body(z_ref, wd1a_ref, wd1b_ref, bd1_ref, wd2_ref, bd2_ref,
                 pos_ref, neg_ref):
    z = z_ref[...]
    zs = z[:B]
    zd = z[B:2 * B]
    zn = z[2 * B:]
    a = jnp.dot(zs, wd1a_ref[...], preferred_element_type=jnp.float32)
    bd1 = bd1_ref[...]
    hp = jnp.maximum(a + jnp.dot(zd, wd1b_ref[...], preferred_element_type=jnp.float32) + bd1, 0.0)
    hn = jnp.maximum(a + jnp.dot(zn, wd1b_ref[...], preferred_element_type=jnp.float32) + bd1, 0.0)
    bd2 = bd2_ref[...]
    pos_ref[...] = jax.nn.sigmoid(jnp.dot(hp, wd2_ref[...], preferred_element_type=jnp.float32) + bd2)
    neg_ref[...] = jax.nn.sigmoid(jnp.dot(hn, wd2_ref[...], preferred_element_type=jnp.float32) + bd2)


def _encode_tc(G3, NF3, PN3, NT, PS, t2c, SS, W1f, W1e, W1t, w1r, b1, W2, b2,
               Wself, t2v_w, t2v_b):
    return pl.pallas_call(
        _encode_body,
        grid=(NBLK,),
        in_specs=[
            pl.BlockSpec((K, SB, F), lambda i: (0, i, 0)),
            pl.BlockSpec((K, SB, EF), lambda i: (0, i, 0)),
            pl.BlockSpec((K, SB, RP), lambda i: (0, i, 0)),
            pl.BlockSpec((SB, K), lambda i: (i, 0)),
            pl.BlockSpec((SB, RP), lambda i: (i, 0)),
            pl.BlockSpec((SB, 1), lambda i: (i, 0)),
            pl.BlockSpec((SB, F), lambda i: (i, 0)),
            pl.BlockSpec((F, H), lambda i: (0, 0)),
            pl.BlockSpec((EF, H), lambda i: (0, 0)),
            pl.BlockSpec((T, H), lambda i: (0, 0)),
            pl.BlockSpec((H,), lambda i: (0,)),
            pl.BlockSpec((H,), lambda i: (0,)),
            pl.BlockSpec((H, H), lambda i: (0, 0)),
            pl.BlockSpec((H,), lambda i: (0,)),
            pl.BlockSpec((F, H), lambda i: (0, 0)),
            pl.BlockSpec((T,), lambda i: (0,)),
            pl.BlockSpec((T,), lambda i: (0,)),
        ],
        out_specs=pl.BlockSpec((SB, H), lambda i: (i, 0)),
        out_shape=jax.ShapeDtypeStruct((S3, H), jnp.float32),
    )(G3, NF3, PN3, NT, PS, t2c, SS, W1f, W1e, W1t, w1r, b1, W2, b2, Wself,
      t2v_w, t2v_b)


def _decode_tc(z, Wd1a, Wd1b, bd1, Wd2, bd2):
    return pl.pallas_call(
        _decode_body,
        in_specs=[
            pl.BlockSpec((S3, H), lambda: (0, 0)),
            pl.BlockSpec((H, H), lambda: (0, 0)),
            pl.BlockSpec((H, H), lambda: (0, 0)),
            pl.BlockSpec((H,), lambda: (0,)),
            pl.BlockSpec((H, 1), lambda: (0, 0)),
            pl.BlockSpec((1,), lambda: (0,)),
        ],
        out_specs=[
            pl.BlockSpec((B, 1), lambda: (0, 0)),
            pl.BlockSpec((B, 1), lambda: (0, 0)),
        ],
        out_shape=[
            jax.ShapeDtypeStruct((B, 1), jnp.float32),
            jax.ShapeDtypeStruct((B, 1), jnp.float32),
        ],
    )(z, Wd1a, Wd1b, bd1, Wd2, bd2)


def kernel(static_node_feat, src, dst, neg, time, nbr_nids, nbr_times,
           nbr_feats, src_nbr_idx, dst_nbr_idx, neg_nbr_idx, t2v_w, t2v_b, P,
           W1, b1, W2, b2, Wself, Wd1, bd1, Wd2, bd2):
    seeds = jnp.concatenate([src, dst, neg]).astype(jnp.int32)
    idx_all = jnp.concatenate(
        [src_nbr_idx, dst_nbr_idx, neg_nbr_idx]).astype(jnp.int32)
    t2 = jnp.concatenate([time, time, time])

    nn_g = jnp.take(nbr_nids, idx_all, axis=0).astype(jnp.int32)
    nn_pre = nn_g.reshape(NW, S_PT, K).transpose(0, 2, 1).reshape(RT)
    ie_g = idx_all[:, None] * K + jnp.arange(K, dtype=jnp.int32)[None, :]
    ie_pre = ie_g.reshape(NW, S_PT, K).transpose(0, 2, 1).reshape(RT)
    nfsrc = nbr_feats.reshape(RT, EF)

    G, SS = _sc_wide_fn()(static_node_feat, nn_pre, seeds)
    PN, NF, PS, NT = _sc_narrow_fn()(P, nfsrc, nn_pre, ie_pre, seeds,
                                     idx_all, nbr_times)

    W1f = W1[:F]
    W1e = W1[F:F + EF]
    W1t = W1[F + EF:F + EF + T]
    w1r = W1[F + EF + T]

    z = _encode_tc(G.reshape(K, S3, F),
                   NF.reshape(K, S3, EF),
                   PN.reshape(K, S3, RP),
                   NT, PS, t2[:, None], SS, W1f, W1e, W1t, w1r,
                   b1, W2, b2, Wself, t2v_w, t2v_b)
    pos2, neg2 = _decode_tc(z, Wd1[:H], Wd1[H:], bd1, Wd2, bd2)
    return (pos2[:, 0], neg2[:, 0])


# speedup vs baseline: 3.6973x; 1.5491x over previous
import functools

import jax
import jax.numpy as jnp
from jax import lax
from jax.experimental import pallas as pl
from jax.experimental.pallas import tpu as pltpu
from jax.experimental.pallas import tpu_sc as plsc

N = 100000
B = 1024
K = 32
F = 128
EF = 16
T = 100
RP = 16
H = 128

S3 = 3 * B
RT = S3 * K
NW = 32
S_PT = S3 // NW

_SC_MESH = dict(core_axis_name="c", subcore_axis_name="s")


def _wid():
    return lax.axis_index("s") * 2 + lax.axis_index("c")


def _sc_wide_body(static_hbm, nn_hbm, seeds_hbm, g_out, ss_out,
                  nnv, sv, ssv, gb0, gb1, semg0, semg1, semw0, semw1, sems):
    wid = _wid()
    base_s = wid * S_PT

    pltpu.sync_copy(nn_hbm.at[pl.ds(wid * (K * S_PT), K * S_PT)], nnv)
    pltpu.sync_copy(seeds_hbm.at[pl.ds(base_s, S_PT)], sv)

    pltpu.async_copy(static_hbm.at[sv], ssv, sems).wait()
    pltpu.sync_copy(ssv, ss_out.at[pl.ds(base_s, S_PT)])

    gbs = (gb0, gb1)
    semg = (semg0, semg1)
    semw = (semw0, semw1)
    dg = [None] * K
    dw = [None] * K
    dg[0] = pltpu.async_copy(static_hbm.at[nnv.at[pl.ds(0, S_PT)]],
                             gbs[0], semg[0])
    for j in range(K):
        b = j & 1
        if j + 1 < K:
            if j >= 1:
                dw[j - 1].wait()
            dg[j + 1] = pltpu.async_copy(
                static_hbm.at[nnv.at[pl.ds((j + 1) * S_PT, S_PT)]],
                gbs[b ^ 1], semg[b ^ 1])
        dg[j].wait()
        dw[j] = pltpu.async_copy(
            gbs[b], g_out.at[pl.ds(j * S3 + base_s, S_PT)], semw[b])
    dw[K - 2].wait()
    dw[K - 1].wait()


@functools.lru_cache(maxsize=1)
def _sc_wide_fn():
    return functools.partial(
        pl.kernel,
        out_type=[
            jax.ShapeDtypeStruct((RT, F), jnp.float32),
            jax.ShapeDtypeStruct((S3, F), jnp.float32),
        ],
        mesh=plsc.VectorSubcoreMesh(**_SC_MESH),
        scratch_types=[
            pltpu.VMEM((K * S_PT,), jnp.int32),
            pltpu.VMEM((S_PT,), jnp.int32),
            pltpu.VMEM((S_PT, F), jnp.float32),
            pltpu.VMEM((S_PT, F), jnp.float32),
            pltpu.VMEM((S_PT, F), jnp.float32),
            pltpu.SemaphoreType.DMA,
            pltpu.SemaphoreType.DMA,
            pltpu.SemaphoreType.DMA,
            pltpu.SemaphoreType.DMA,
            pltpu.SemaphoreType.DMA,
        ],
    )(_sc_wide_body)


def _sc_narrow_body(p_hbm, nfsrc_hbm, nn_hbm, ie_hbm, seeds_hbm, idx_hbm,
                    nt_hbm, pn_out, nf_out, ps_out, nt_out,
                    nnv, iev, sv, iv, psv, ntv, pnall, nfall,
                    semp, semf, sems, semw):
    wid = _wid()
    base_s = wid * S_PT

    pltpu.sync_copy(nn_hbm.at[pl.ds(wid * (K * S_PT), K * S_PT)], nnv)
    pltpu.sync_copy(ie_hbm.at[pl.ds(wid * (K * S_PT), K * S_PT)], iev)
    pltpu.sync_copy(seeds_hbm.at[pl.ds(base_s, S_PT)], sv)
    pltpu.sync_copy(idx_hbm.at[pl.ds(base_s, S_PT)], iv)

    dps = pltpu.async_copy(p_hbm.at[sv], psv, sems)
    dnt = pltpu.async_copy(nt_hbm.at[iv], ntv, sems)
    dp = []
    df = []
    for j in range(K):
        sl = pl.ds(j * S_PT, S_PT)
        dp.append(pltpu.async_copy(p_hbm.at[nnv.at[sl]],
                                   pnall.at[sl], semp))
        df.append(pltpu.async_copy(nfsrc_hbm.at[iev.at[sl]],
                                   nfall.at[sl], semf))
    dps.wait()
    dnt.wait()
    dws = [pltpu.async_copy(psv, ps_out.at[pl.ds(base_s, S_PT)], semw),
           pltpu.async_copy(ntv, nt_out.at[pl.ds(base_s, S_PT)], semw)]
    for j in range(K):
        sl = pl.ds(j * S_PT, S_PT)
        dp[j].wait()
        dws.append(pltpu.async_copy(
            pnall.at[sl], pn_out.at[pl.ds(j * S3 + base_s, S_PT)], semw))
        df[j].wait()
        dws.append(pltpu.async_copy(
            nfall.at[sl], nf_out.at[pl.ds(j * S3 + base_s, S_PT)], semw))
    for d in dws:
        d.wait()


@functools.lru_cache(maxsize=1)
def _sc_narrow_fn():
    return functools.partial(
        pl.kernel,
        out_type=[
            jax.ShapeDtypeStruct((K * S3, RP), jnp.float32),
            jax.ShapeDtypeStruct((K * S3, EF), jnp.float32),
            jax.ShapeDtypeStruct((S3, RP), jnp.float32),
            jax.ShapeDtypeStruct((S3, K), jnp.float32),
        ],
        mesh=plsc.VectorSubcoreMesh(**_SC_MESH),
        compiler_params=pltpu.CompilerParams(use_tc_tiling_on_sc=False),
        scratch_types=[
            pltpu.VMEM((K * S_PT,), jnp.int32),
            pltpu.VMEM((K * S_PT,), jnp.int32),
            pltpu.VMEM((S_PT,), jnp.int32),
            pltpu.VMEM((S_PT,), jnp.int32),
            pltpu.VMEM((S_PT, RP), jnp.float32),
            pltpu.VMEM((S_PT, K), jnp.float32),
            pltpu.VMEM((K * S_PT, RP), jnp.float32),
            pltpu.VMEM((K * S_PT, EF), jnp.float32),
            pltpu.SemaphoreType.DMA,
            pltpu.SemaphoreType.DMA,
            pltpu.SemaphoreType.DMA,
            pltpu.SemaphoreType.DMA,
        ],
    )(_sc_narrow_body)


SB = 256
RB = SB * K
NBLK = S3 // SB


_INV2PI = 0.15915494309189535
_RND = 12582912.0
_P2HI = 6.2831855
_P2LO = -1.7484555e-07
_COS_C = (1.0, -0.49999988, 0.04166649, -0.0013887803, 2.4769883e-05,
          -2.707903e-07, 1.7245092e-09)


def _fast_cos(x):
    n = jnp.round(x * _INV2PI)
    r = x - n * _P2HI
    r = r - n * _P2LO
    u = r * r
    acc = _COS_C[6]
    for k in range(5, -1, -1):
        acc = acc * u + _COS_C[k]
    return acc


def _encode_body(g_ref, nf_ref, pn_ref, nt_ref, ps_ref, t2_ref, ss_ref,
                 w1f_ref, w1e_ref, w1t_ref, w1r_ref, b1_ref, w2_ref, b2_ref,
                 wself_ref, tw_ref, tb_ref, z_ref):
    nt = nt_ref[...]
    t2 = t2_ref[...]
    ps = ps_ref[...]
    g3 = g_ref[...]
    nf3 = nf_ref[...]
    pn3 = pn_ref[...]

    dtm = t2 - nt
    dt_full = jnp.concatenate([dtm[:, j:j + 1] for j in range(K)], axis=0)
    te = _fast_cos(dt_full * tw_ref[...] + tb_ref[...])
    g2 = jnp.concatenate([g3[j] for j in range(K)], axis=0)
    nf2 = jnp.concatenate([nf3[j] for j in range(K)], axis=0)
    rp = jnp.concatenate(
        [jnp.sum(pn3[j] * ps, axis=-1, keepdims=True) for j in range(K)],
        axis=0)

    pre = (jnp.dot(g2, w1f_ref[...], preferred_element_type=jnp.float32)
           + jnp.dot(nf2, w1e_ref[...], preferred_element_type=jnp.float32)
           + jnp.dot(te, w1t_ref[...], preferred_element_type=jnp.float32)
           + rp * w1r_ref[...]
           + b1_ref[...])
    h = jnp.maximum(pre, 0.0)
    acc = h[:SB]
    for j in range(1, K):
        acc = acc + h[j * SB:(j + 1) * SB]
    m = acc * (1.0 / K)
    z = (jnp.dot(m, w2_ref[...], preferred_element_type=jnp.float32)
         + b2_ref[...]
         + jnp.dot(ss_ref[...], wself_ref[...], preferred_element_type=jnp.float32))
    z_ref[...] = z


def _decode_body(z_ref, wd1a_ref, wd1b_ref, bd1_ref, wd2_ref, bd2_ref,
                 pos_ref, neg_ref):
    z = z_ref[...]
    zs = z[:B]
    zd = z[B:2 * B]
    zn = z[2 * B:]
    a = jnp.dot(zs, wd1a_ref[...], preferred_element_type=jnp.float32)
    bd1 = bd1_ref[...]
    hp = jnp.maximum(a + jnp.dot(zd, wd1b_ref[...], preferred_element_type=jnp.float32) + bd1, 0.0)
    hn = jnp.maximum(a + jnp.dot(zn, wd1b_ref[...], preferred_element_type=jnp.float32) + bd1, 0.0)
    bd2 = bd2_ref[...]
    pos_ref[...] = jax.nn.sigmoid(jnp.dot(hp, wd2_ref[...], preferred_element_type=jnp.float32) + bd2)
    neg_ref[...] = jax.nn.sigmoid(jnp.dot(hn, wd2_ref[...], preferred_element_type=jnp.float32) + bd2)


def _encode_tc(G3, NF3, PN3, NT, PS, t2c, SS, W1f, W1e, W1t, w1r, b1, W2, b2,
               Wself, t2v_w, t2v_b):
    return pl.pallas_call(
        _encode_body,
        grid=(NBLK,),
        in_specs=[
            pl.BlockSpec((K, SB, F), lambda i: (0, i, 0)),
            pl.BlockSpec((K, SB, EF), lambda i: (0, i, 0)),
            pl.BlockSpec((K, SB, RP), lambda i: (0, i, 0)),
            pl.BlockSpec((SB, K), lambda i: (i, 0)),
            pl.BlockSpec((SB, RP), lambda i: (i, 0)),
            pl.BlockSpec((SB, 1), lambda i: (i, 0)),
            pl.BlockSpec((SB, F), lambda i: (i, 0)),
            pl.BlockSpec((F, H), lambda i: (0, 0)),
            pl.BlockSpec((EF, H), lambda i: (0, 0)),
            pl.BlockSpec((T, H), lambda i: (0, 0)),
            pl.BlockSpec((H,), lambda i: (0,)),
            pl.BlockSpec((H,), lambda i: (0,)),
            pl.BlockSpec((H, H), lambda i: (0, 0)),
            pl.BlockSpec((H,), lambda i: (0,)),
            pl.BlockSpec((F, H), lambda i: (0, 0)),
            pl.BlockSpec((T,), lambda i: (0,)),
            pl.BlockSpec((T,), lambda i: (0,)),
        ],
        out_specs=pl.BlockSpec((SB, H), lambda i: (i, 0)),
        out_shape=jax.ShapeDtypeStruct((S3, H), jnp.float32),
    )(G3, NF3, PN3, NT, PS, t2c, SS, W1f, W1e, W1t, w1r, b1, W2, b2, Wself,
      t2v_w, t2v_b)


def _decode_tc(z, Wd1a, Wd1b, bd1, Wd2, bd2):
    return pl.pallas_call(
        _decode_body,
        in_specs=[
            pl.BlockSpec((S3, H), lambda: (0, 0)),
            pl.BlockSpec((H, H), lambda: (0, 0)),
            pl.BlockSpec((H, H), lambda: (0, 0)),
            pl.BlockSpec((H,), lambda: (0,)),
            pl.BlockSpec((H, 1), lambda: (0, 0)),
            pl.BlockSpec((1,), lambda: (0,)),
        ],
        out_specs=[
            pl.BlockSpec((B, 1), lambda: (0, 0)),
            pl.BlockSpec((B, 1), lambda: (0, 0)),
        ],
        out_shape=[
            jax.ShapeDtypeStruct((B, 1), jnp.float32),
            jax.ShapeDtypeStruct((B, 1), jnp.float32),
        ],
    )(z, Wd1a, Wd1b, bd1, Wd2, bd2)


def kernel(static_node_feat, src, dst, neg, time, nbr_nids, nbr_times,
           nbr_feats, src_nbr_idx, dst_nbr_idx, neg_nbr_idx, t2v_w, t2v_b, P,
           W1, b1, W2, b2, Wself, Wd1, bd1, Wd2, bd2):
    seeds = jnp.concatenate([src, dst, neg]).astype(jnp.int32)
    idx_all = jnp.concatenate(
        [src_nbr_idx, dst_nbr_idx, neg_nbr_idx]).astype(jnp.int32)
    t2 = jnp.concatenate([time, time, time])

    nn_g = jnp.take(nbr_nids, idx_all, axis=0).astype(jnp.int32)
    nn_pre = nn_g.reshape(NW, S_PT, K).transpose(0, 2, 1).reshape(RT)
    ie_g = idx_all[:, None] * K + jnp.arange(K, dtype=jnp.int32)[None, :]
    ie_pre = ie_g.reshape(NW, S_PT, K).transpose(0, 2, 1).reshape(RT)
    nfsrc = nbr_feats.reshape(RT, EF)

    G, SS = _sc_wide_fn()(static_node_feat, nn_pre, seeds)
    PN, NF, PS, NT = _sc_narrow_fn()(P, nfsrc, nn_pre, ie_pre, seeds,
                                     idx_all, nbr_times)

    W1f = W1[:F]
    W1e = W1[F:F + EF]
    W1t = W1[F + EF:F + EF + T]
    w1r = W1[F + EF + T]

    z = _encode_tc(G.reshape(K, S3, F),
                   NF.reshape(K, S3, EF),
                   PN.reshape(K, S3, RP),
                   NT, PS, t2[:, None], SS, W1f, W1e, W1t, w1r,
                   b1, W2, b2, Wself, t2v_w, t2v_b)
    pos2, neg2 = _decode_tc(z, Wd1[:H], Wd1[H:], bd1, Wd2, bd2)
    return (pos2[:, 0], neg2[:, 0])


# speedup vs baseline: 4.7012x; 1.2715x over previous
import functools

import jax
import jax.numpy as jnp
from jax import lax
from jax.experimental import pallas as pl
from jax.experimental.pallas import tpu as pltpu
from jax.experimental.pallas import tpu_sc as plsc

N = 100000
B = 1024
K = 32
F = 128
EF = 16
T = 100
RP = 16
H = 128

S3 = 3 * B
RT = S3 * K
NW = 32
S_PT = S3 // NW

_SC_MESH = dict(core_axis_name="c", subcore_axis_name="s")


def _wid():
    return lax.axis_index("s") * 2 + lax.axis_index("c")


def _sc_wide_body(static_hbm, nf2_hbm, nn_hbm, seeds_hbm, idx_hbm,
                  g_out, ss_out, nf_out,
                  nnv, sv, iv, ssv, nfv, gb0, gb1,
                  semg0, semg1, semw0, semw1, sems):
    wid = _wid()
    base_s = wid * S_PT

    pltpu.sync_copy(nn_hbm.at[pl.ds(wid * (K * S_PT), K * S_PT)], nnv)
    pltpu.sync_copy(seeds_hbm.at[pl.ds(base_s, S_PT)], sv)
    pltpu.sync_copy(idx_hbm.at[pl.ds(base_s, S_PT)], iv)

    dnf = pltpu.async_copy(nf2_hbm.at[iv], nfv, semg1)
    pltpu.async_copy(static_hbm.at[sv], ssv, sems).wait()
    pltpu.sync_copy(ssv, ss_out.at[pl.ds(base_s, S_PT)])
    dnf.wait()
    pltpu.sync_copy(nfv, nf_out.at[pl.ds(base_s, S_PT)])

    gbs = (gb0, gb1)
    semg = (semg0, semg1)
    semw = (semw0, semw1)
    dg = [None] * K
    dw = [None] * K
    dg[0] = pltpu.async_copy(static_hbm.at[nnv.at[pl.ds(0, S_PT)]],
                             gbs[0], semg[0])
    for j in range(K):
        b = j & 1
        if j + 1 < K:
            if j >= 1:
                dw[j - 1].wait()
            dg[j + 1] = pltpu.async_copy(
                static_hbm.at[nnv.at[pl.ds((j + 1) * S_PT, S_PT)]],
                gbs[b ^ 1], semg[b ^ 1])
        dg[j].wait()
        dw[j] = pltpu.async_copy(
            gbs[b], g_out.at[pl.ds(j * S3 + base_s, S_PT)], semw[b])
    dw[K - 2].wait()
    dw[K - 1].wait()


@functools.lru_cache(maxsize=1)
def _sc_wide_fn():
    return functools.partial(
        pl.kernel,
        out_type=[
            jax.ShapeDtypeStruct((RT, F), jnp.float32),
            jax.ShapeDtypeStruct((S3, F), jnp.float32),
            jax.ShapeDtypeStruct((S3, K * EF), jnp.float32),
        ],
        mesh=plsc.VectorSubcoreMesh(**_SC_MESH),
        scratch_types=[
            pltpu.VMEM((K * S_PT,), jnp.int32),
            pltpu.VMEM((S_PT,), jnp.int32),
            pltpu.VMEM((S_PT,), jnp.int32),
            pltpu.VMEM((S_PT, F), jnp.float32),
            pltpu.VMEM((S_PT, K * EF), jnp.float32),
            pltpu.VMEM((S_PT, F), jnp.float32),
            pltpu.VMEM((S_PT, F), jnp.float32),
            pltpu.SemaphoreType.DMA,
            pltpu.SemaphoreType.DMA,
            pltpu.SemaphoreType.DMA,
            pltpu.SemaphoreType.DMA,
            pltpu.SemaphoreType.DMA,
        ],
    )(_sc_wide_body)


def _sc_narrow_body(p_hbm, nn_hbm, seeds_hbm, idx_hbm, nt_hbm, t2_hbm,
                    rp_out, dt_out,
                    nnv, sv, iv, t2v, psv, ntv, dtv, pnall, rpv,
                    semp, sems, semw):
    wid = _wid()
    base_s = wid * S_PT

    pltpu.sync_copy(nn_hbm.at[pl.ds(wid * (K * S_PT), K * S_PT)], nnv)
    pltpu.sync_copy(seeds_hbm.at[pl.ds(base_s, S_PT)], sv)
    pltpu.sync_copy(idx_hbm.at[pl.ds(base_s, S_PT)], iv)
    pltpu.sync_copy(t2_hbm.at[pl.ds(base_s, S_PT)], t2v)

    dps = pltpu.async_copy(p_hbm.at[sv], psv, sems)
    dnt = pltpu.async_copy(nt_hbm.at[iv], ntv, sems)
    dp = []
    for j in range(K):
        sl = pl.ds(j * S_PT, S_PT)
        dp.append(pltpu.async_copy(p_hbm.at[nnv.at[sl]],
                                   pnall.at[sl], semp))
    dps.wait()
    dnt.wait()

    def dt_body(gi, carry):
        t2vec = t2v[pl.ds(gi * 16, 16)]
        for si in range(16):
            s = gi * 16 + si
            for g in range(2):
                sl = pl.ds(g * 16, 16)
                dtv[s, sl] = t2vec[si] - ntv[s, sl]
        return carry

    lax.fori_loop(0, S_PT // 16, dt_body, 0)
    dwd = pltpu.async_copy(dtv, dt_out.at[pl.ds(base_s, S_PT)], semw)

    for j in range(K):
        dp[j].wait()

    lanes = lax.iota(jnp.int32, 16)

    def rp_body(gi, carry):
        for si in range(16):
            s = gi * 16 + si
            psrow = psv[s, :]
            for g in range(2):
                rows = (lanes + (g * 16)) * S_PT + s
                acc = jnp.zeros((16,), jnp.float32)
                for d in range(RP):
                    v = plsc.load_gather(
                        pnall, [rows, jnp.full((16,), d, jnp.int32)])
                    acc = acc + v * psrow[d]
                rpv[s, pl.ds(g * 16, 16)] = acc
        return carry

    lax.fori_loop(0, S_PT // 16, rp_body, 0)
    dwr = pltpu.async_copy(rpv, rp_out.at[pl.ds(base_s, S_PT)], semw)
    dwd.wait()
    dwr.wait()


@functools.lru_cache(maxsize=1)
def _sc_narrow_fn():
    return functools.partial(
        pl.kernel,
        out_type=[
            jax.ShapeDtypeStruct((S3, K), jnp.float32),
            jax.ShapeDtypeStruct((S3, K), jnp.float32),
        ],
        mesh=plsc.VectorSubcoreMesh(**_SC_MESH),
        compiler_params=pltpu.CompilerParams(use_tc_tiling_on_sc=False,
                                             needs_layout_passes=False),
        scratch_types=[
            pltpu.VMEM((K * S_PT,), jnp.int32),
            pltpu.VMEM((S_PT,), jnp.int32),
            pltpu.VMEM((S_PT,), jnp.int32),
            pltpu.VMEM((S_PT,), jnp.float32),
            pltpu.VMEM((S_PT, RP), jnp.float32),
            pltpu.VMEM((S_PT, K), jnp.float32),
            pltpu.VMEM((S_PT, K), jnp.float32),
            pltpu.VMEM((K * S_PT, RP), jnp.float32),
            pltpu.VMEM((S_PT, K), jnp.float32),
            pltpu.SemaphoreType.DMA,
            pltpu.SemaphoreType.DMA,
            pltpu.SemaphoreType.DMA,
        ],
    )(_sc_narrow_body)


SB = 256
RB = SB * K
NBLK = S3 // SB


_INV2PI = 0.15915494309189535
_RND = 12582912.0
_P2HI = 6.2831855
_P2LO = -1.7484555e-07
_COS_C = (1.0, -0.49999988, 0.04166649, -0.0013887803, 2.4769883e-05,
          -2.707903e-07, 1.7245092e-09)


def _fast_cos(x):
    n = jnp.round(x * _INV2PI)
    r = x - n * _P2HI
    r = r - n * _P2LO
    u = r * r
    acc = _COS_C[6]
    for k in range(5, -1, -1):
        acc = acc * u + _COS_C[k]
    return acc


def _encode_body(g_ref, nf_ref, dt_ref, rp_ref, ss_ref,
                 w1f_ref, w1e_ref, w1t_ref, w1r_ref, b1_ref, w2_ref, b2_ref,
                 wself_ref, tw_ref, tb_ref, z_ref):
    dtm = dt_ref[...]
    rpm = rp_ref[...]
    g3 = g_ref[...]
    nfw = nf_ref[...]

    dt_full = jnp.concatenate([dtm[:, j:j + 1] for j in range(K)], axis=0)
    te = _fast_cos(dt_full * tw_ref[...] + tb_ref[...])
    g2 = jnp.concatenate([g3[j] for j in range(K)], axis=0)
    nf2 = jnp.concatenate(
        [nfw[:, j * EF:(j + 1) * EF] for j in range(K)], axis=0)
    rp = jnp.concatenate(
        [rpm[:, j:j + 1] for j in range(K)], axis=0)

    pre = (jnp.dot(g2, w1f_ref[...], preferred_element_type=jnp.float32)
           + jnp.dot(nf2, w1e_ref[...], preferred_element_type=jnp.float32)
           + jnp.dot(te, w1t_ref[...], preferred_element_type=jnp.float32)
           + rp * w1r_ref[...]
           + b1_ref[...])
    h = jnp.maximum(pre, 0.0)
    acc = h[:SB]
    for j in range(1, K):
        acc = acc + h[j * SB:(j + 1) * SB]
    m = acc * (1.0 / K)
    z = (jnp.dot(m, w2_ref[...], preferred_element_type=jnp.float32)
         + b2_ref[...]
         + jnp.dot(ss_ref[...], wself_ref[...], preferred_element_type=jnp.float32))
    z_ref[...] = z


def _decode_body(z_ref, wd1a_ref, wd1b_ref, bd1_ref, wd2_ref, bd2_ref,
                 pos_ref, neg_ref):
    z = z_ref[...]
    zs = z[:B]
    zd = z[B:2 * B]
    zn = z[2 * B:]
    a = jnp.dot(zs, wd1a_ref[...], preferred_element_type=jnp.float32)
    bd1 = bd1_ref[...]
    hp = jnp.maximum(a + jnp.dot(zd, wd1b_ref[...], preferred_element_type=jnp.float32) + bd1, 0.0)
    hn = jnp.maximum(a + jnp.dot(zn, wd1b_ref[...], preferred_element_type=jnp.float32) + bd1, 0.0)
    bd2 = bd2_ref[...]
    pos_ref[...] = jax.nn.sigmoid(jnp.dot(hp, wd2_ref[...], preferred_element_type=jnp.float32) + bd2)
    neg_ref[...] = jax.nn.sigmoid(jnp.dot(hn, wd2_ref[...], preferred_element_type=jnp.float32) + bd2)


def _encode_tc(G3, NF3, DT, RPm, SS, W1f, W1e, W1t, w1r, b1, W2, b2,
               Wself, t2v_w, t2v_b):
    return pl.pallas_call(
        _encode_body,
        grid=(NBLK,),
        in_specs=[
            pl.BlockSpec((K, SB, F), lambda i: (0, i, 0)),
            pl.BlockSpec((SB, K * EF), lambda i: (i, 0)),
            pl.BlockSpec((SB, K), lambda i: (i, 0)),
            pl.BlockSpec((SB, K), lambda i: (i, 0)),
            pl.BlockSpec((SB, F), lambda i: (i, 0)),
            pl.BlockSpec((F, H), lambda i: (0, 0)),
            pl.BlockSpec((EF, H), lambda i: (0, 0)),
            pl.BlockSpec((T, H), lambda i: (0, 0)),
            pl.BlockSpec((H,), lambda i: (0,)),
            pl.BlockSpec((H,), lambda i: (0,)),
            pl.BlockSpec((H, H), lambda i: (0, 0)),
            pl.BlockSpec((H,), lambda i: (0,)),
            pl.BlockSpec((F, H), lambda i: (0, 0)),
            pl.BlockSpec((T,), lambda i: (0,)),
            pl.BlockSpec((T,), lambda i: (0,)),
        ],
        out_specs=pl.BlockSpec((SB, H), lambda i: (i, 0)),
        out_shape=jax.ShapeDtypeStruct((S3, H), jnp.float32),
    )(G3, NF3, DT, RPm, SS, W1f, W1e, W1t, w1r, b1, W2, b2, Wself,
      t2v_w, t2v_b)


def _decode_tc(z, Wd1a, Wd1b, bd1, Wd2, bd2):
    return pl.pallas_call(
        _decode_body,
        in_specs=[
            pl.BlockSpec((S3, H), lambda: (0, 0)),
            pl.BlockSpec((H, H), lambda: (0, 0)),
            pl.BlockSpec((H, H), lambda: (0, 0)),
            pl.BlockSpec((H,), lambda: (0,)),
            pl.BlockSpec((H, 1), lambda: (0, 0)),
            pl.BlockSpec((1,), lambda: (0,)),
        ],
        out_specs=[
            pl.BlockSpec((B, 1), lambda: (0, 0)),
            pl.BlockSpec((B, 1), lambda: (0, 0)),
        ],
        out_shape=[
            jax.ShapeDtypeStruct((B, 1), jnp.float32),
            jax.ShapeDtypeStruct((B, 1), jnp.float32),
        ],
    )(z, Wd1a, Wd1b, bd1, Wd2, bd2)


def kernel(static_node_feat, src, dst, neg, time, nbr_nids, nbr_times,
           nbr_feats, src_nbr_idx, dst_nbr_idx, neg_nbr_idx, t2v_w, t2v_b, P,
           W1, b1, W2, b2, Wself, Wd1, bd1, Wd2, bd2):
    seeds = jnp.concatenate([src, dst, neg]).astype(jnp.int32)
    idx_all = jnp.concatenate(
        [src_nbr_idx, dst_nbr_idx, neg_nbr_idx]).astype(jnp.int32)
    t2 = jnp.concatenate([time, time, time])

    nn_g = jnp.take(nbr_nids, idx_all, axis=0).astype(jnp.int32)
    nn_pre = nn_g.reshape(NW, S_PT, K).transpose(0, 2, 1).reshape(RT)
    nf2 = nbr_feats.reshape(S3, K * EF)

    G, SS, NF = _sc_wide_fn()(static_node_feat, nf2, nn_pre, seeds, idx_all)
    RPm, DT = _sc_narrow_fn()(P, nn_pre, seeds, idx_all, nbr_times, t2)

    W1f = W1[:F]
    W1e = W1[F:F + EF]
    W1t = W1[F + EF:F + EF + T]
    w1r = W1[F + EF + T]

    z = _encode_tc(G.reshape(K, S3, F), NF, DT, RPm, SS, W1f, W1e, W1t, w1r,
                   b1, W2, b2, Wself, t2v_w, t2v_b)
    pos2, neg2 = _decode_tc(z, Wd1[:H], Wd1[H:], bd1, Wd2, bd2)
    return (pos2[:, 0], neg2[:, 0])


# speedup vs baseline: 5.2139x; 1.1090x over previous
import functools

import jax
import jax.numpy as jnp
from jax import lax
from jax.experimental import pallas as pl
from jax.experimental.pallas import tpu as pltpu
from jax.experimental.pallas import tpu_sc as plsc

N = 100000
B = 1024
K = 32
F = 128
EF = 16
T = 100
RP = 16
H = 128

S3 = 3 * B
RT = S3 * K
NW = 32
S_PT = S3 // NW

_SC_MESH = dict(core_axis_name="c", subcore_axis_name="s")


def _wid():
    return lax.axis_index("s") * 2 + lax.axis_index("c")


def _sc_wide_body(static_hbm, nf2_hbm, nn_hbm, seeds_hbm, idx_hbm,
                  g_out, ss_out, nf_out,
                  nnv, sv, iv, ssv, nfv, gb0, gb1,
                  semg0, semg1, semw0, semw1, sems):
    wid = _wid()
    base_s = wid * S_PT

    pltpu.sync_copy(nn_hbm.at[pl.ds(wid * (K * S_PT), K * S_PT)], nnv)
    pltpu.sync_copy(seeds_hbm.at[pl.ds(base_s, S_PT)], sv)
    pltpu.sync_copy(idx_hbm.at[pl.ds(base_s, S_PT)], iv)

    dnf = pltpu.async_copy(nf2_hbm.at[iv], nfv, semg1)
    pltpu.async_copy(static_hbm.at[sv], ssv, sems).wait()
    pltpu.sync_copy(ssv, ss_out.at[pl.ds(base_s, S_PT)])
    dnf.wait()
    pltpu.sync_copy(nfv, nf_out.at[pl.ds(base_s, S_PT)])

    gbs = (gb0, gb1)
    semg = (semg0, semg1)
    semw = (semw0, semw1)
    dg = [None] * K
    dw = [None] * K
    dg[0] = pltpu.async_copy(static_hbm.at[nnv.at[pl.ds(0, S_PT)]],
                             gbs[0], semg[0])
    for j in range(K):
        b = j & 1
        if j + 1 < K:
            if j >= 1:
                dw[j - 1].wait()
            dg[j + 1] = pltpu.async_copy(
                static_hbm.at[nnv.at[pl.ds((j + 1) * S_PT, S_PT)]],
                gbs[b ^ 1], semg[b ^ 1])
        dg[j].wait()
        dw[j] = pltpu.async_copy(
            gbs[b], g_out.at[pl.ds(j * S3 + base_s, S_PT)], semw[b])
    dw[K - 2].wait()
    dw[K - 1].wait()


@functools.lru_cache(maxsize=1)
def _sc_wide_fn():
    return functools.partial(
        pl.kernel,
        out_type=[
            jax.ShapeDtypeStruct((RT, F), jnp.float32),
            jax.ShapeDtypeStruct((S3, F), jnp.float32),
            jax.ShapeDtypeStruct((S3, K * EF), jnp.float32),
        ],
        mesh=plsc.VectorSubcoreMesh(**_SC_MESH),
        scratch_types=[
            pltpu.VMEM((K * S_PT,), jnp.int32),
            pltpu.VMEM((S_PT,), jnp.int32),
            pltpu.VMEM((S_PT,), jnp.int32),
            pltpu.VMEM((S_PT, F), jnp.float32),
            pltpu.VMEM((S_PT, K * EF), jnp.float32),
            pltpu.VMEM((S_PT, F), jnp.float32),
            pltpu.VMEM((S_PT, F), jnp.float32),
            pltpu.SemaphoreType.DMA,
            pltpu.SemaphoreType.DMA,
            pltpu.SemaphoreType.DMA,
            pltpu.SemaphoreType.DMA,
            pltpu.SemaphoreType.DMA,
        ],
    )(_sc_wide_body)


def _sc_narrow_body(p_hbm, nn_hbm, seeds_hbm, idx_hbm, nt_hbm, t2_hbm,
                    rp_out, dt_out,
                    nnv, sv, iv, t2v, psv, ntv, dtv, pnall, rpv,
                    semp, sems, semw):
    wid = _wid()
    base_s = wid * S_PT

    pltpu.sync_copy(nn_hbm.at[pl.ds(wid * (K * S_PT), K * S_PT)], nnv)
    pltpu.sync_copy(seeds_hbm.at[pl.ds(base_s, S_PT)], sv)
    pltpu.sync_copy(idx_hbm.at[pl.ds(base_s, S_PT)], iv)
    pltpu.sync_copy(t2_hbm.at[pl.ds(base_s, S_PT)], t2v)

    dps = pltpu.async_copy(p_hbm.at[sv], psv, sems)
    dnt = pltpu.async_copy(nt_hbm.at[iv], ntv, sems)
    dp = []
    for j in range(K):
        sl = pl.ds(j * S_PT, S_PT)
        dp.append(pltpu.async_copy(p_hbm.at[nnv.at[sl]],
                                   pnall.at[sl], semp))
    dps.wait()
    dnt.wait()

    def dt_body(gi, carry):
        t2vec = t2v[pl.ds(gi * 16, 16)]
        for si in range(16):
            s = gi * 16 + si
            for g in range(2):
                sl = pl.ds(g * 16, 16)
                dtv[s, sl] = t2vec[si] - ntv[s, sl]
        return carry

    lax.fori_loop(0, S_PT // 16, dt_body, 0)
    dwd = pltpu.async_copy(dtv, dt_out.at[pl.ds(base_s, S_PT)], semw)

    for j in range(K):
        dp[j].wait()

    lanes = lax.iota(jnp.int32, 16)

    def rp_body(gi, carry):
        for si in range(16):
            s = gi * 16 + si
            psrow = psv[s, :]
            for g in range(2):
                rows = (lanes + (g * 16)) * S_PT + s
                acc = jnp.zeros((16,), jnp.float32)
                for d in range(RP):
                    v = plsc.load_gather(
                        pnall, [rows, jnp.full((16,), d, jnp.int32)])
                    acc = acc + v * psrow[d]
                rpv[s, pl.ds(g * 16, 16)] = acc
        return carry

    lax.fori_loop(0, S_PT // 16, rp_body, 0)
    dwr = pltpu.async_copy(rpv, rp_out.at[pl.ds(base_s, S_PT)], semw)
    dwd.wait()
    dwr.wait()


@functools.lru_cache(maxsize=1)
def _sc_narrow_fn():
    return functools.partial(
        pl.kernel,
        out_type=[
            jax.ShapeDtypeStruct((S3, K), jnp.float32),
            jax.ShapeDtypeStruct((S3, K), jnp.float32),
        ],
        mesh=plsc.VectorSubcoreMesh(**_SC_MESH),
        compiler_params=pltpu.CompilerParams(use_tc_tiling_on_sc=False,
                                             needs_layout_passes=False),
        scratch_types=[
            pltpu.VMEM((K * S_PT,), jnp.int32),
            pltpu.VMEM((S_PT,), jnp.int32),
            pltpu.VMEM((S_PT,), jnp.int32),
            pltpu.VMEM((S_PT,), jnp.float32),
            pltpu.VMEM((S_PT, RP), jnp.float32),
            pltpu.VMEM((S_PT, K), jnp.float32),
            pltpu.VMEM((S_PT, K), jnp.float32),
            pltpu.VMEM((K * S_PT, RP), jnp.float32),
            pltpu.VMEM((S_PT, K), jnp.float32),
            pltpu.SemaphoreType.DMA,
            pltpu.SemaphoreType.DMA,
            pltpu.SemaphoreType.DMA,
        ],
    )(_sc_narrow_body)


SB = 256
RB = SB * K
NBLK = S3 // SB


_INV2PI = 0.15915494309189535
_RND = 12582912.0
_P2HI = 6.2831855
_P2LO = -1.7484555e-07
_COS_C = (1.0, -0.49999988, 0.04166649, -0.0013887803, 2.4769883e-05,
          -2.707903e-07, 1.7245092e-09)


def _fast_cos(x):
    n = jnp.round(x * _INV2PI)
    r = x - n * _P2HI
    r = r - n * _P2LO
    u = r * r
    acc = _COS_C[6]
    for k in range(5, -1, -1):
        acc = acc * u + _COS_C[k]
    return acc


def _encode_body(g_ref, nf_ref, dt_ref, rp_ref, ss_ref,
                 w1f_ref, w1e_ref, w1t_ref, w1r_ref, b1_ref, w2_ref, b2_ref,
                 wself_ref, tw_ref, tb_ref, z_ref):
    dtm = dt_ref[...]
    rpm = rp_ref[...]
    g3 = g_ref[...]
    nfw = nf_ref[...]
    w1f = w1f_ref[...]
    w1e = w1e_ref[...]
    w1t = w1t_ref[...]
    w1r = w1r_ref[...]
    b1 = b1_ref[...]
    tw = tw_ref[...]
    tb = tb_ref[...]

    acc = jnp.zeros((SB, H), jnp.float32)
    for j in range(K):
        te_j = _fast_cos(dtm[:, j:j + 1] * tw + tb)
        pre_j = (jnp.dot(g3[j], w1f, preferred_element_type=jnp.float32)
                 + jnp.dot(nfw[:, j * EF:(j + 1) * EF], w1e,
                           preferred_element_type=jnp.float32)
                 + jnp.dot(te_j, w1t, preferred_element_type=jnp.float32)
                 + rpm[:, j:j + 1] * w1r
                 + b1)
        acc = acc + jnp.maximum(pre_j, 0.0)
    m = acc * (1.0 / K)
    z = (jnp.dot(m, w2_ref[...], preferred_element_type=jnp.float32)
         + b2_ref[...]
         + jnp.dot(ss_ref[...], wself_ref[...], preferred_element_type=jnp.float32))
    z_ref[...] = z


def _decode_body(z_ref, wd1a_ref, wd1b_ref, bd1_ref, wd2_ref, bd2_ref,
                 pos_ref, neg_ref):
    z = z_ref[...]
    zs = z[:B]
    zd = z[B:2 * B]
    zn = z[2 * B:]
    a = jnp.dot(zs, wd1a_ref[...], preferred_element_type=jnp.float32)
    bd1 = bd1_ref[...]
    hp = jnp.maximum(a + jnp.dot(zd, wd1b_ref[...], preferred_element_type=jnp.float32) + bd1, 0.0)
    hn = jnp.maximum(a + jnp.dot(zn, wd1b_ref[...], preferred_element_type=jnp.float32) + bd1, 0.0)
    bd2 = bd2_ref[...]
    pos_ref[...] = jax.nn.sigmoid(jnp.dot(hp, wd2_ref[...], preferred_element_type=jnp.float32) + bd2)
    neg_ref[...] = jax.nn.sigmoid(jnp.dot(hn, wd2_ref[...], preferred_element_type=jnp.float32) + bd2)


def _encode_tc(G3, NF3, DT, RPm, SS, W1f, W1e, W1t, w1r, b1, W2, b2,
               Wself, t2v_w, t2v_b):
    return pl.pallas_call(
        _encode_body,
        grid=(NBLK,),
        in_specs=[
            pl.BlockSpec((K, SB, F), lambda i: (0, i, 0)),
            pl.BlockSpec((SB, K * EF), lambda i: (i, 0)),
            pl.BlockSpec((SB, K), lambda i: (i, 0)),
            pl.BlockSpec((SB, K), lambda i: (i, 0)),
            pl.BlockSpec((SB, F), lambda i: (i, 0)),
            pl.BlockSpec((F, H), lambda i: (0, 0)),
            pl.BlockSpec((EF, H), lambda i: (0, 0)),
            pl.BlockSpec((T, H), lambda i: (0, 0)),
            pl.BlockSpec((H,), lambda i: (0,)),
            pl.BlockSpec((H,), lambda i: (0,)),
            pl.BlockSpec((H, H), lambda i: (0, 0)),
            pl.BlockSpec((H,), lambda i: (0,)),
            pl.BlockSpec((F, H), lambda i: (0, 0)),
            pl.BlockSpec((T,), lambda i: (0,)),
            pl.BlockSpec((T,), lambda i: (0,)),
        ],
        out_specs=pl.BlockSpec((SB, H), lambda i: (i, 0)),
        out_shape=jax.ShapeDtypeStruct((S3, H), jnp.float32),
    )(G3, NF3, DT, RPm, SS, W1f, W1e, W1t, w1r, b1, W2, b2, Wself,
      t2v_w, t2v_b)


def _decode_tc(z, Wd1a, Wd1b, bd1, Wd2, bd2):
    return pl.pallas_call(
        _decode_body,
        in_specs=[
            pl.BlockSpec((S3, H), lambda: (0, 0)),
            pl.BlockSpec((H, H), lambda: (0, 0)),
            pl.BlockSpec((H, H), lambda: (0, 0)),
            pl.BlockSpec((H,), lambda: (0,)),
            pl.BlockSpec((H, 1), lambda: (0, 0)),
            pl.BlockSpec((1,), lambda: (0,)),
        ],
        out_specs=[
            pl.BlockSpec((B, 1), lambda: (0, 0)),
            pl.BlockSpec((B, 1), lambda: (0, 0)),
        ],
        out_shape=[
            jax.ShapeDtypeStruct((B, 1), jnp.float32),
            jax.ShapeDtypeStruct((B, 1), jnp.float32),
        ],
    )(z, Wd1a, Wd1b, bd1, Wd2, bd2)


def kernel(static_node_feat, src, dst, neg, time, nbr_nids, nbr_times,
           nbr_feats, src_nbr_idx, dst_nbr_idx, neg_nbr_idx, t2v_w, t2v_b, P,
           W1, b1, W2, b2, Wself, Wd1, bd1, Wd2, bd2):
    seeds = jnp.concatenate([src, dst, neg]).astype(jnp.int32)
    idx_all = jnp.concatenate(
        [src_nbr_idx, dst_nbr_idx, neg_nbr_idx]).astype(jnp.int32)
    t2 = jnp.concatenate([time, time, time])

    nn_g = jnp.take(nbr_nids, idx_all, axis=0).astype(jnp.int32)
    nn_pre = nn_g.reshape(NW, S_PT, K).transpose(0, 2, 1).reshape(RT)
    nf2 = nbr_feats.reshape(S3, K * EF)

    G, SS, NF = _sc_wide_fn()(static_node_feat, nf2, nn_pre, seeds, idx_all)
    RPm, DT = _sc_narrow_fn()(P, nn_pre, seeds, idx_all, nbr_times, t2)

    W1f = W1[:F]
    W1e = W1[F:F + EF]
    W1t = W1[F + EF:F + EF + T]
    w1r = W1[F + EF + T]

    z = _encode_tc(G.reshape(K, S3, F), NF, DT, RPm, SS, W1f, W1e, W1t, w1r,
                   b1, W2, b2, Wself, t2v_w, t2v_b)
    pos2, neg2 = _decode_tc(z, Wd1[:H], Wd1[H:], bd1, Wd2, bd2)
    return (pos2[:, 0], neg2[:, 0])


# speedup vs baseline: 5.3559x; 1.0272x over previous
import functools

import jax
import jax.numpy as jnp
from jax import lax
from jax.experimental import pallas as pl
from jax.experimental.pallas import tpu as pltpu
from jax.experimental.pallas import tpu_sc as plsc

N = 100000
B = 1024
K = 32
F = 128
EF = 16
T = 100
RP = 16
H = 128

S3 = 3 * B
RT = S3 * K
NW = 32
S_PT = S3 // NW

_SC_MESH = dict(core_axis_name="c", subcore_axis_name="s")


def _wid():
    return lax.axis_index("s") * 2 + lax.axis_index("c")


def _sc_wide_body(static_hbm, nf2_hbm, nn_hbm, seeds_hbm, idx_hbm,
                  g_out, ss_out, nf_out,
                  nnv, sv, iv, ssv, nfv, gb0, gb1,
                  semg0, semg1, semw0, semw1, sems):
    wid = _wid()
    base_s = wid * S_PT

    pltpu.sync_copy(nn_hbm.at[pl.ds(wid * (K * S_PT), K * S_PT)], nnv)
    pltpu.sync_copy(seeds_hbm.at[pl.ds(base_s, S_PT)], sv)
    pltpu.sync_copy(idx_hbm.at[pl.ds(base_s, S_PT)], iv)

    dnf = pltpu.async_copy(nf2_hbm.at[iv], nfv, semg1)
    pltpu.async_copy(static_hbm.at[sv], ssv, sems).wait()
    pltpu.sync_copy(ssv, ss_out.at[pl.ds(base_s, S_PT)])
    dnf.wait()
    pltpu.sync_copy(nfv, nf_out.at[pl.ds(base_s, S_PT)])

    gbs = (gb0, gb1)
    semg = (semg0, semg1)
    semw = (semw0, semw1)
    dg = [None] * K
    dw = [None] * K
    dg[0] = pltpu.async_copy(static_hbm.at[nnv.at[pl.ds(0, S_PT)]],
                             gbs[0], semg[0])
    for j in range(K):
        b = j & 1
        if j + 1 < K:
            if j >= 1:
                dw[j - 1].wait()
            dg[j + 1] = pltpu.async_copy(
                static_hbm.at[nnv.at[pl.ds((j + 1) * S_PT, S_PT)]],
                gbs[b ^ 1], semg[b ^ 1])
        dg[j].wait()
        dw[j] = pltpu.async_copy(
            gbs[b], g_out.at[pl.ds(j * S3 + base_s, S_PT)], semw[b])
    dw[K - 2].wait()
    dw[K - 1].wait()


@functools.lru_cache(maxsize=1)
def _sc_wide_fn():
    return functools.partial(
        pl.kernel,
        out_type=[
            jax.ShapeDtypeStruct((RT, F), jnp.float32),
            jax.ShapeDtypeStruct((S3, F), jnp.float32),
            jax.ShapeDtypeStruct((S3, K * EF), jnp.float32),
        ],
        mesh=plsc.VectorSubcoreMesh(**_SC_MESH),
        scratch_types=[
            pltpu.VMEM((K * S_PT,), jnp.int32),
            pltpu.VMEM((S_PT,), jnp.int32),
            pltpu.VMEM((S_PT,), jnp.int32),
            pltpu.VMEM((S_PT, F), jnp.float32),
            pltpu.VMEM((S_PT, K * EF), jnp.float32),
            pltpu.VMEM((S_PT, F), jnp.float32),
            pltpu.VMEM((S_PT, F), jnp.float32),
            pltpu.SemaphoreType.DMA,
            pltpu.SemaphoreType.DMA,
            pltpu.SemaphoreType.DMA,
            pltpu.SemaphoreType.DMA,
            pltpu.SemaphoreType.DMA,
        ],
    )(_sc_wide_body)


def _sc_narrow_body(p_hbm, nids_hbm, seeds_hbm, idx_hbm, nt_hbm, t2_hbm,
                    rp_out, dt_out, nn_out,
                    nnjm, nnrows, sv, iv, t2v, psv, ntv, dtv, pnall, rpv,
                    semp, sems, semw):
    wid = _wid()
    base_s = wid * S_PT

    pltpu.sync_copy(seeds_hbm.at[pl.ds(base_s, S_PT)], sv)
    pltpu.sync_copy(idx_hbm.at[pl.ds(base_s, S_PT)], iv)
    pltpu.sync_copy(t2_hbm.at[pl.ds(base_s, S_PT)], t2v)

    pltpu.async_copy(nids_hbm.at[iv], nnrows, sems).wait()
    lanes = lax.iota(jnp.int32, 16)
    for j in range(K):
        jfull = jnp.full((16,), j, jnp.int32)
        for g in range(S_PT // 16):
            v = plsc.load_gather(nnrows, [lanes + (g * 16), jfull])
            nnjm[pl.ds(j * S_PT + g * 16, 16)] = v
    dnn = pltpu.async_copy(nnjm, nn_out.at[pl.ds(wid * (K * S_PT), K * S_PT)],
                           semw)

    dps = pltpu.async_copy(p_hbm.at[sv], psv, sems)
    dnt = pltpu.async_copy(nt_hbm.at[iv], ntv, sems)
    dp = []
    for j in range(K):
        sl = pl.ds(j * S_PT, S_PT)
        dp.append(pltpu.async_copy(p_hbm.at[nnjm.at[sl]],
                                   pnall.at[sl], semp))
    dps.wait()
    dnt.wait()

    def dt_body(gi, carry):
        t2vec = t2v[pl.ds(gi * 16, 16)]
        for si in range(16):
            s = gi * 16 + si
            for g in range(2):
                sl = pl.ds(g * 16, 16)
                dtv[s, sl] = t2vec[si] - ntv[s, sl]
        return carry

    lax.fori_loop(0, S_PT // 16, dt_body, 0)
    dwd = pltpu.async_copy(dtv, dt_out.at[pl.ds(base_s, S_PT)], semw)

    for j in range(K):
        dp[j].wait()

    lanes = lax.iota(jnp.int32, 16)

    def rp_body(gi, carry):
        for si in range(16):
            s = gi * 16 + si
            psrow = psv[s, :]
            for g in range(2):
                rows = (lanes + (g * 16)) * S_PT + s
                acc = jnp.zeros((16,), jnp.float32)
                for d in range(RP):
                    v = plsc.load_gather(
                        pnall, [rows, jnp.full((16,), d, jnp.int32)])
                    acc = acc + v * psrow[d]
                rpv[s, pl.ds(g * 16, 16)] = acc
        return carry

    lax.fori_loop(0, S_PT // 16, rp_body, 0)
    dwr = pltpu.async_copy(rpv, rp_out.at[pl.ds(base_s, S_PT)], semw)
    dnn.wait()
    dwd.wait()
    dwr.wait()


@functools.lru_cache(maxsize=1)
def _sc_narrow_fn():
    return functools.partial(
        pl.kernel,
        out_type=[
            jax.ShapeDtypeStruct((S3, K), jnp.float32),
            jax.ShapeDtypeStruct((S3, K), jnp.float32),
            jax.ShapeDtypeStruct((RT,), jnp.int32),
        ],
        mesh=plsc.VectorSubcoreMesh(**_SC_MESH),
        compiler_params=pltpu.CompilerParams(use_tc_tiling_on_sc=False,
                                             needs_layout_passes=False),
        scratch_types=[
            pltpu.VMEM((K * S_PT,), jnp.int32),
            pltpu.VMEM((S_PT, K), jnp.int32),
            pltpu.VMEM((S_PT,), jnp.int32),
            pltpu.VMEM((S_PT,), jnp.int32),
            pltpu.VMEM((S_PT,), jnp.float32),
            pltpu.VMEM((S_PT, RP), jnp.float32),
            pltpu.VMEM((S_PT, K), jnp.float32),
            pltpu.VMEM((S_PT, K), jnp.float32),
            pltpu.VMEM((K * S_PT, RP), jnp.float32),
            pltpu.VMEM((S_PT, K), jnp.float32),
            pltpu.SemaphoreType.DMA,
            pltpu.SemaphoreType.DMA,
            pltpu.SemaphoreType.DMA,
        ],
    )(_sc_narrow_body)


SB = 256
RB = SB * K
NBLK = S3 // SB


_INV2PI = 0.15915494309189535
_RND = 12582912.0
_P2HI = 6.2831855
_P2LO = -1.7484555e-07
_COS_C = (1.0, -0.49999988, 0.04166649, -0.0013887803, 2.4769883e-05,
          -2.707903e-07, 1.7245092e-09)


def _fast_cos(x):
    n = jnp.round(x * _INV2PI)
    r = x - n * _P2HI
    r = r - n * _P2LO
    u = r * r
    acc = _COS_C[6]
    for k in range(5, -1, -1):
        acc = acc * u + _COS_C[k]
    return acc


def _encode_body(g_ref, nf_ref, dt_ref, rp_ref, ss_ref,
                 w1f_ref, w1e_ref, w1t_ref, w1r_ref, b1_ref, w2_ref, b2_ref,
                 wself_ref, tw_ref, tb_ref, z_ref):
    dtm = dt_ref[...]
    rpm = rp_ref[...]
    g3 = g_ref[...]
    nfw = nf_ref[...]
    w1f = w1f_ref[...]
    w1e = w1e_ref[...]
    w1t = w1t_ref[...]
    w1r = w1r_ref[...]
    b1 = b1_ref[...]
    tw = tw_ref[...]
    tb = tb_ref[...]

    acc = jnp.zeros((SB, H), jnp.float32)
    for j in range(K):
        te_j = _fast_cos(dtm[:, j:j + 1] * tw + tb)
        pre_j = (jnp.dot(g3[j], w1f, preferred_element_type=jnp.float32)
                 + jnp.dot(nfw[:, j * EF:(j + 1) * EF], w1e,
                           preferred_element_type=jnp.float32)
                 + jnp.dot(te_j, w1t, preferred_element_type=jnp.float32)
                 + rpm[:, j:j + 1] * w1r
                 + b1)
        acc = acc + jnp.maximum(pre_j, 0.0)
    m = acc * (1.0 / K)
    z = (jnp.dot(m, w2_ref[...], preferred_element_type=jnp.float32)
         + b2_ref[...]
         + jnp.dot(ss_ref[...], wself_ref[...], preferred_element_type=jnp.float32))
    z_ref[...] = z


def _decode_body(z_ref, wd1a_ref, wd1b_ref, bd1_ref, wd2_ref, bd2_ref,
                 pos_ref, neg_ref):
    z = z_ref[...]
    zs = z[:B]
    zd = z[B:2 * B]
    zn = z[2 * B:]
    a = jnp.dot(zs, wd1a_ref[...], preferred_element_type=jnp.float32)
    bd1 = bd1_ref[...]
    hp = jnp.maximum(a + jnp.dot(zd, wd1b_ref[...], preferred_element_type=jnp.float32) + bd1, 0.0)
    hn = jnp.maximum(a + jnp.dot(zn, wd1b_ref[...], preferred_element_type=jnp.float32) + bd1, 0.0)
    bd2 = bd2_ref[...]
    pos_ref[...] = jax.nn.sigmoid(jnp.dot(hp, wd2_ref[...], preferred_element_type=jnp.float32) + bd2)
    neg_ref[...] = jax.nn.sigmoid(jnp.dot(hn, wd2_ref[...], preferred_element_type=jnp.float32) + bd2)


def _encode_tc(G3, NF3, DT, RPm, SS, W1f, W1e, W1t, w1r, b1, W2, b2,
               Wself, t2v_w, t2v_b):
    return pl.pallas_call(
        _encode_body,
        grid=(NBLK,),
        in_specs=[
            pl.BlockSpec((K, SB, F), lambda i: (0, i, 0)),
            pl.BlockSpec((SB, K * EF), lambda i: (i, 0)),
            pl.BlockSpec((SB, K), lambda i: (i, 0)),
            pl.BlockSpec((SB, K), lambda i: (i, 0)),
            pl.BlockSpec((SB, F), lambda i: (i, 0)),
            pl.BlockSpec((F, H), lambda i: (0, 0)),
            pl.BlockSpec((EF, H), lambda i: (0, 0)),
            pl.BlockSpec((T, H), lambda i: (0, 0)),
            pl.BlockSpec((H,), lambda i: (0,)),
            pl.BlockSpec((H,), lambda i: (0,)),
            pl.BlockSpec((H, H), lambda i: (0, 0)),
            pl.BlockSpec((H,), lambda i: (0,)),
            pl.BlockSpec((F, H), lambda i: (0, 0)),
            pl.BlockSpec((T,), lambda i: (0,)),
            pl.BlockSpec((T,), lambda i: (0,)),
        ],
        out_specs=pl.BlockSpec((SB, H), lambda i: (i, 0)),
        out_shape=jax.ShapeDtypeStruct((S3, H), jnp.float32),
    )(G3, NF3, DT, RPm, SS, W1f, W1e, W1t, w1r, b1, W2, b2, Wself,
      t2v_w, t2v_b)


def _decode_tc(z, Wd1a, Wd1b, bd1, Wd2, bd2):
    return pl.pallas_call(
        _decode_body,
        in_specs=[
            pl.BlockSpec((S3, H), lambda: (0, 0)),
            pl.BlockSpec((H, H), lambda: (0, 0)),
            pl.BlockSpec((H, H), lambda: (0, 0)),
            pl.BlockSpec((H,), lambda: (0,)),
            pl.BlockSpec((H, 1), lambda: (0, 0)),
            pl.BlockSpec((1,), lambda: (0,)),
        ],
        out_specs=[
            pl.BlockSpec((B, 1), lambda: (0, 0)),
            pl.BlockSpec((B, 1), lambda: (0, 0)),
        ],
        out_shape=[
            jax.ShapeDtypeStruct((B, 1), jnp.float32),
            jax.ShapeDtypeStruct((B, 1), jnp.float32),
        ],
    )(z, Wd1a, Wd1b, bd1, Wd2, bd2)


def kernel(static_node_feat, src, dst, neg, time, nbr_nids, nbr_times,
           nbr_feats, src_nbr_idx, dst_nbr_idx, neg_nbr_idx, t2v_w, t2v_b, P,
           W1, b1, W2, b2, Wself, Wd1, bd1, Wd2, bd2):
    seeds = jnp.concatenate([src, dst, neg]).astype(jnp.int32)
    idx_all = jnp.concatenate(
        [src_nbr_idx, dst_nbr_idx, neg_nbr_idx]).astype(jnp.int32)
    t2 = jnp.concatenate([time, time, time])

    nf2 = nbr_feats.reshape(S3, K * EF)

    RPm, DT, NNJM = _sc_narrow_fn()(P, nbr_nids.astype(jnp.int32), seeds,
                                    idx_all, nbr_times, t2)
    G, SS, NF = _sc_wide_fn()(static_node_feat, nf2, NNJM, seeds, idx_all)

    W1f = W1[:F]
    W1e = W1[F:F + EF]
    W1t = W1[F + EF:F + EF + T]
    w1r = W1[F + EF + T]

    z = _encode_tc(G.reshape(K, S3, F), NF, DT, RPm, SS, W1f, W1e, W1t, w1r,
                   b1, W2, b2, Wself, t2v_w, t2v_b)
    pos2, neg2 = _decode_tc(z, Wd1[:H], Wd1[H:], bd1, Wd2, bd2)
    return (pos2[:, 0], neg2[:, 0])
